# baseline (speedup 1.0000x reference)
import jax
import jax.numpy as jnp
from jax.experimental import pallas as pl


def _chamfer_body(q, n, p_ref, t_ref, chamfer_ref, idx_ref):
    p = p_ref[...]
    t = t_ref[...]
    p2 = jnp.sum(p * p, axis=1, keepdims=True)
    t2 = jnp.sum(t * t, axis=1, keepdims=True)
    ones_q = jnp.ones((p.shape[0], 1), jnp.float32)
    ones_n = jnp.ones((t.shape[0], 1), jnp.float32)
    p_aug = jnp.concatenate([-2.0 * p, p2, ones_q], axis=1)
    t_aug = jnp.concatenate([t, ones_n, t2], axis=1)
    d2 = jax.lax.dot_general(
        p_aug, t_aug, (((1,), (1,)), ((), ())),
        preferred_element_type=jnp.float32,
        precision=jax.lax.Precision.HIGHEST)

    colmin = jnp.min(d2, axis=0, keepdims=True)
    iota = jax.lax.broadcasted_iota(
        jnp.int32, (d2.shape[0], 1), 0).astype(jnp.float32)
    idxf = jnp.min(jnp.where(d2 == colmin, iota, jnp.float32(3e38)),
                   axis=0, keepdims=True)
    idx_ref[...] = idxf.astype(jnp.int32)

    col_d2 = jnp.maximum(colmin, 0.0)
    rowmin = jnp.min(d2, axis=1, keepdims=True)
    row_d2 = jnp.maximum(rowmin, 0.0)
    chamfer = (jnp.sum(jnp.sqrt(row_d2)) / q
               + jnp.sum(jnp.sqrt(col_d2)) / n)
    chamfer_ref[...] = jnp.full((1, 1), chamfer, jnp.float32)


def kernel(pred_points, tgt_points):
    bs, q, d = pred_points.shape
    n = tgt_points.shape[0] * tgt_points.shape[1]
    p = pred_points.reshape(q, d)
    t = tgt_points.reshape(n, d)

    chamfer2d, idx2d = pl.pallas_call(
        lambda *refs: _chamfer_body(q, n, *refs),
        out_shape=[
            jax.ShapeDtypeStruct((1, 1), jnp.float32),
            jax.ShapeDtypeStruct((1, n), jnp.int32),
        ],
    )(p, t)

    return chamfer2d[0, 0], idx2d


# speedup vs baseline: 1.0014x; 1.0014x over previous
import jax
import jax.numpy as jnp
from jax.experimental import pallas as pl
from jax.experimental.pallas import tpu as pltpu


def _chamfer_body(q, n, p_ref, t_ref, chamfer_ref, idx_ref):
    p = p_ref[...]
    t = t_ref[...]
    p2 = jnp.sum(p * p, axis=1, keepdims=True)
    t2 = jnp.sum(t * t, axis=1, keepdims=True)
    ones_q = jnp.ones((p.shape[0], 1), jnp.float32)
    ones_n = jnp.ones((t.shape[0], 1), jnp.float32)
    p_aug = jnp.concatenate([-2.0 * p, p2, ones_q], axis=1)
    t_aug = jnp.concatenate([t, ones_n, t2], axis=1)
    d2 = jax.lax.dot_general(
        p_aug, t_aug, (((1,), (1,)), ((), ())),
        preferred_element_type=jnp.float32,
        precision=jax.lax.Precision.HIGHEST)

    colmin = jnp.min(d2, axis=0, keepdims=True)
    iota = jax.lax.broadcasted_iota(
        jnp.int32, (d2.shape[0], 1), 0).astype(jnp.float32)
    idxf = jnp.min(jnp.where(d2 == colmin, iota, jnp.float32(3e38)),
                   axis=0, keepdims=True)
    idx_ref[...] = idxf.astype(jnp.int32)

    col_d2 = jnp.maximum(colmin, 0.0)
    rowmin = jnp.min(d2, axis=1, keepdims=True)
    row_d2 = jnp.maximum(rowmin, 0.0)
    chamfer = (jnp.sum(jnp.sqrt(row_d2)) / q
               + jnp.sum(jnp.sqrt(col_d2)) / n)
    chamfer_ref[...] = jnp.full((1, 1), chamfer, jnp.float32)


def kernel(pred_points, tgt_points):
    bs, q, d = pred_points.shape
    n = tgt_points.shape[0] * tgt_points.shape[1]
    p = pred_points.reshape(q, d)
    t = tgt_points.reshape(n, d)

    chamfer2d, idx2d = pl.pallas_call(
        lambda *refs: _chamfer_body(q, n, *refs),
        out_shape=[
            jax.ShapeDtypeStruct((1, 1), jnp.float32),
            jax.ShapeDtypeStruct((1, n), jnp.int32),
        ],
        compiler_params=pltpu.CompilerParams(
            skip_device_barrier=True,
            disable_bounds_checks=True,
            disable_semaphore_checks=True,
        ),
    )(p, t)

    return chamfer2d[0, 0], idx2d


# speedup vs baseline: 1.0020x; 1.0006x over previous
import jax
import jax.numpy as jnp
from jax.experimental import pallas as pl


def _chamfer_body(q, n, p_ref, t_ref, chamfer_ref, idx_ref):
    p = p_ref[...]
    t = t_ref[...]
    p2 = jnp.sum(p * p, axis=1, keepdims=True)
    t2 = jnp.sum(t * t, axis=1, keepdims=True)
    ones_q = jnp.ones((p.shape[0], 1), jnp.float32)
    ones_n = jnp.ones((t.shape[0], 1), jnp.float32)
    p_aug = jnp.concatenate([-2.0 * p, p2, ones_q], axis=1)
    t_aug = jnp.concatenate([t, ones_n, t2], axis=1)
    d2 = jax.lax.dot_general(
        p_aug, t_aug, (((1,), (1,)), ((), ())),
        preferred_element_type=jnp.float32,
        precision=jax.lax.Precision.HIGHEST)

    colmin = jnp.min(d2, axis=0, keepdims=True)
    iota = jax.lax.broadcasted_iota(
        jnp.int32, (d2.shape[0], 1), 0).astype(jnp.float32)
    idxf = jnp.min(jnp.where(d2 == colmin, iota, jnp.float32(3e38)),
                   axis=0, keepdims=True)
    idx_ref[...] = idxf.astype(jnp.int32)

    col_d2 = jnp.maximum(colmin, 0.0)
    rowmin = jnp.min(d2, axis=1, keepdims=True)
    row_d2 = jnp.maximum(rowmin, 0.0)
    chamfer = (jnp.sum(jnp.sqrt(row_d2)) / q
               + jnp.sum(jnp.sqrt(col_d2)) / n)
    chamfer_ref[...] = jnp.full((1, 1), chamfer, jnp.float32)


def kernel(pred_points, tgt_points):
    bs, q, d = pred_points.shape
    n = tgt_points.shape[0] * tgt_points.shape[1]
    p = pred_points.reshape(q, d)
    t = tgt_points.reshape(n, d)

    chamfer2d, idx2d = pl.pallas_call(
        lambda *refs: _chamfer_body(q, n, *refs),
        out_shape=[
            jax.ShapeDtypeStruct((1, 1), jnp.float32),
            jax.ShapeDtypeStruct((1, n), jnp.int32),
        ],
    )(p, t)

    return chamfer2d[0, 0], idx2d
